# baseline (speedup 1.0000x reference)
import functools
import math

import jax
import jax.numpy as jnp
import numpy as np
from jax import lax
from jax.experimental import pallas as pl
from jax.experimental.pallas import tpu as pltpu
from jax.experimental.pallas import tpu_sc as plsc

D = 256
H = 8
DH = D // H
N = 10000
E = 160000

NC = 2
NS = 16
EPW = E // (NC * NS)
GC = 40
SCC = 200
EPT = E // NS
NCH = EPT // SCC
RPT = N // NS

@functools.cache
def _mesh():
    return plsc.VectorSubcoreMesh(core_axis_name="c", subcore_axis_name="s")


def _ea_body(et_ref, w_ref, b_ref, o_ref):
    o_ref[...] = jnp.dot(et_ref[...], w_ref[...],
                         preferred_element_type=jnp.float32) + b_ref[...]


def _edge_attr(et, w, b):
    BE = 2000
    return pl.pallas_call(
        _ea_body,
        grid=(E // BE,),
        in_specs=[pl.BlockSpec((BE, 16), lambda i: (i, 0)),
                  pl.BlockSpec((16, D), lambda i: (0, 0)),
                  pl.BlockSpec((1, D), lambda i: (0, 0))],
        out_specs=pl.BlockSpec((BE, D), lambda i: (i, 0)),
        out_shape=jax.ShapeDtypeStruct((E, D), jnp.float32),
    )(et, w, b.reshape(1, D))


def _proj_body(h_ref, wq_ref, bq_ref, wk_ref, bk_ref, wv_ref, bv_ref,
               a_ref, m_ref, q_ref, kv_ref):
    h = h_ref[...]
    q_ref[...] = jnp.dot(h, wq_ref[...],
                         preferred_element_type=jnp.float32) + bq_ref[...]
    k = jnp.dot(h, wk_ref[...], preferred_element_type=jnp.float32) + bk_ref[...]
    v = jnp.dot(h, wv_ref[...], preferred_element_type=jnp.float32) + bv_ref[...]
    kv_ref[:, :D] = jnp.dot(k, a_ref[...], preferred_element_type=jnp.float32)
    kv_ref[:, D:] = jnp.dot(v, m_ref[...], preferred_element_type=jnp.float32)


def _proj(h, wq, bq, wk, bk, wv, bv, a_bd, m_bd):
    BN = 1000
    return pl.pallas_call(
        _proj_body,
        grid=(N // BN,),
        in_specs=[pl.BlockSpec((BN, D), lambda i: (i, 0))] +
                 [pl.BlockSpec((D, D), lambda i: (0, 0)),
                  pl.BlockSpec((1, D), lambda i: (0, 0))] * 3 +
                 [pl.BlockSpec((D, D), lambda i: (0, 0)),
                  pl.BlockSpec((D, D), lambda i: (0, 0))],
        out_specs=[pl.BlockSpec((BN, D), lambda i: (i, 0)),
                   pl.BlockSpec((BN, 2 * D), lambda i: (i, 0))],
        out_shape=[jax.ShapeDtypeStruct((N, D), jnp.float32),
                   jax.ShapeDtypeStruct((N, 2 * D), jnp.float32)],
    )(h, wq, bq.reshape(1, D), wk, bk.reshape(1, D), wv, bv.reshape(1, D),
      a_bd, m_bd)


def _edge_math_body(q_ref, kv_ref, ea_ref, sc_ref, g_ref, gt_ref,
                    contrib_ref, w_ref):
    ea = ea_ref[...]
    kj = kv_ref[:, :D] + ea
    vj = kv_ref[:, D:] + ea
    prod = q_ref[...] * kj * sc_ref[...]
    alpha = jnp.dot(prod, g_ref[...], preferred_element_type=jnp.float32)
    w = jnp.exp(alpha)
    wrep = jnp.dot(w, gt_ref[...], preferred_element_type=jnp.float32)
    c = vj * wrep
    contrib_ref[0] = c[:, :128]
    contrib_ref[1] = c[:, 128:]
    w_ref[0] = wrep[:, :128]
    w_ref[1] = wrep[:, 128:]


def _edge_math(qe, kve, ea, scale, g, gt):
    BE = 2000
    return pl.pallas_call(
        _edge_math_body,
        grid=(E // BE,),
        in_specs=[pl.BlockSpec((BE, D), lambda i: (i, 0)),
                  pl.BlockSpec((BE, 2 * D), lambda i: (i, 0)),
                  pl.BlockSpec((BE, D), lambda i: (i, 0)),
                  pl.BlockSpec((1, D), lambda i: (0, 0)),
                  pl.BlockSpec((D, H), lambda i: (0, 0)),
                  pl.BlockSpec((H, D), lambda i: (0, 0))],
        out_specs=[pl.BlockSpec((NC, BE, 128), lambda i: (0, i, 0)),
                   pl.BlockSpec((NC, BE, 128), lambda i: (0, i, 0))],
        out_shape=[jax.ShapeDtypeStruct((NC, E, 128), jnp.float32),
                   jax.ShapeDtypeStruct((NC, E, 128), jnp.float32)],
    )(qe, kve, ea, scale, g, gt)


def _final_body(acc_ref, s_ref, h_ref, wa_ref, ba_ref, sk_ref,
                g_ref, b_ref, o_ref):
    accc = jnp.concatenate([acc_ref[0], acc_ref[1]], axis=-1)
    srep = jnp.concatenate([s_ref[0], s_ref[1]], axis=-1)
    o1 = accc / (srep + 1e-9)
    g1 = jax.nn.gelu(o1)
    o2 = jnp.dot(g1, wa_ref[...], preferred_element_type=jnp.float32) + ba_ref[...]
    beta = jax.nn.sigmoid(sk_ref[0, 0])
    o3 = beta * o2 + (1.0 - beta) * h_ref[...]
    mu = jnp.mean(o3, axis=-1, keepdims=True)
    xc = o3 - mu
    var = jnp.mean(xc * xc, axis=-1, keepdims=True)
    o_ref[...] = g_ref[...] * (xc / jnp.sqrt(var + 1e-5)) + b_ref[...]


def _final(acc, sacc, h, wa, ba, skip, g, b):
    BN = 1000
    return pl.pallas_call(
        _final_body,
        grid=(N // BN,),
        in_specs=[pl.BlockSpec((NC, BN, 128), lambda i: (0, i, 0)),
                  pl.BlockSpec((NC, BN, 128), lambda i: (0, i, 0)),
                  pl.BlockSpec((BN, D), lambda i: (i, 0)),
                  pl.BlockSpec((D, D), lambda i: (0, 0)),
                  pl.BlockSpec((1, D), lambda i: (0, 0)),
                  pl.BlockSpec((1, 1), lambda i: (0, 0)),
                  pl.BlockSpec((1, D), lambda i: (0, 0)),
                  pl.BlockSpec((1, D), lambda i: (0, 0))],
        out_specs=pl.BlockSpec((BN, D), lambda i: (i, 0)),
        out_shape=jax.ShapeDtypeStruct((N, D), jnp.float32),
    )(acc, sacc, h, wa, ba.reshape(1, D), skip.reshape(1, 1),
      g.reshape(1, D), b.reshape(1, D))


GNC = EPW // 128
GTL = EPW - GNC * 128


def _gather(kv, q, srcA, srcB, dstA, dstB):
    @functools.partial(
        pl.kernel,
        out_type=[jax.ShapeDtypeStruct((E, 2 * D), jnp.float32),
                  jax.ShapeDtypeStruct((E, D), jnp.float32)],
        mesh=_mesh(),
        scratch_types=[pltpu.VMEM((GNC, 128), jnp.int32),
                       pltpu.VMEM((GNC, 128), jnp.int32),
                       pltpu.VMEM((16,), jnp.int32),
                       pltpu.VMEM((16,), jnp.int32),
                       pltpu.VMEM((128, 2 * D), jnp.float32),
                       pltpu.VMEM((128, D), jnp.float32),
                       pltpu.SemaphoreType.DMA,
                       pltpu.SemaphoreType.DMA,
                       pltpu.SemaphoreType.DMA,
                       pltpu.SemaphoreType.DMA],
    )
    def k(kv_hbm, q_hbm, srcA_hbm, srcB_hbm, dstA_hbm, dstB_hbm,
          kve_hbm, qe_hbm, sidx, didx, sidxb, didxb, kvbuf, qbuf,
          wsem_kv, wsem_q, gsem_kv, gsem_q):
        wid = lax.axis_index("c") * NS + lax.axis_index("s")
        base = pl.multiple_of(wid * EPW, 8)
        pltpu.sync_copy(srcA_hbm.at[wid], sidx)
        pltpu.sync_copy(dstA_hbm.at[wid], didx)
        pltpu.sync_copy(srcB_hbm.at[wid], sidxb)
        pltpu.sync_copy(dstB_hbm.at[wid], didxb)

        @pl.loop(0, GNC)
        def _(i):
            off = pl.multiple_of(i * 128, 8)

            @pl.when(i > 0)
            def _():
                pltpu.make_async_copy(
                    kvbuf, kve_hbm.at[pl.ds(base, 128)], wsem_kv).wait()
                pltpu.make_async_copy(
                    qbuf, qe_hbm.at[pl.ds(base, 128)], wsem_q).wait()
            pltpu.async_copy(kv_hbm.at[sidx.at[i]], kvbuf, gsem_kv)
            pltpu.async_copy(q_hbm.at[didx.at[i]], qbuf, gsem_q)
            pltpu.make_async_copy(kv_hbm.at[sidx.at[i]], kvbuf, gsem_kv).wait()
            pltpu.async_copy(kvbuf, kve_hbm.at[pl.ds(base + off, 128)],
                             wsem_kv)
            pltpu.make_async_copy(q_hbm.at[didx.at[i]], qbuf, gsem_q).wait()
            pltpu.async_copy(qbuf, qe_hbm.at[pl.ds(base + off, 128)], wsem_q)

        pltpu.make_async_copy(kvbuf, kve_hbm.at[pl.ds(base, 128)],
                              wsem_kv).wait()
        pltpu.make_async_copy(qbuf, qe_hbm.at[pl.ds(base, 128)],
                              wsem_q).wait()

        tb = pl.multiple_of(base + EPW - 16, 8)
        pltpu.sync_copy(kv_hbm.at[sidxb], kvbuf.at[pl.ds(0, 16)])
        pltpu.sync_copy(kvbuf.at[pl.ds(0, 16)], kve_hbm.at[pl.ds(tb, 16)])
        pltpu.sync_copy(q_hbm.at[didxb], qbuf.at[pl.ds(0, 16)])
        pltpu.sync_copy(qbuf.at[pl.ds(0, 16)], qe_hbm.at[pl.ds(tb, 16)])

    return k(kv, q, srcA, srcB, dstA, dstB)


SNC = EPT // 128
STL = EPT - SNC * 128


def _scatter(x, dstA, dstB):
    @functools.partial(
        pl.kernel,
        out_type=jax.ShapeDtypeStruct((NC, N, 128), jnp.float32),
        mesh=_mesh(),
        scratch_types=[pltpu.VMEM((128,), jnp.int32),
                       pltpu.VMEM((128,), jnp.int32),
                       pltpu.VMEM((16,), jnp.int32),
                       pltpu.VMEM((128, 128), jnp.float32),
                       pltpu.VMEM((128, 128), jnp.float32),
                       pltpu.VMEM_SHARED((N, 128), jnp.float32),
                       pltpu.SemaphoreType.DMA,
                       pltpu.SemaphoreType.DMA],
    )
    def k(c_hbm, dA_hbm, dB_hbm, acc_hbm, didx0, didx1, didxb,
          cbuf, cbuf1, acc_sh, sem0, sem1):
        c = lax.axis_index("c")
        t = lax.axis_index("s")
        nfull = N // 128
        nper = (nfull + NS - 1) // NS
        pltpu.sync_copy(dB_hbm.at[t], didxb)

        @pl.loop(0, 128)
        def _(r):
            @pl.loop(0, 8)
            def _(j):
                cbuf[r, pl.ds(j * 16, 16)] = jnp.zeros((16,), jnp.float32)

        @pl.loop(0, nper)
        def _(j):
            kk = t + j * NS

            @pl.when(kk < nfull)
            def _():
                off = pl.multiple_of(kk * 128, 8)
                pltpu.sync_copy(cbuf, acc_sh.at[pl.ds(off, 128)])

        @pl.when(t == NS - 1)
        def _():
            pltpu.sync_copy(cbuf.at[pl.ds(0, 16)],
                            acc_sh.at[pl.ds(nfull * 128, 16)])
        plsc.subcore_barrier()

        ebase = t * EPT

        def _load(i, dslot, cslot):
            off = pl.multiple_of(ebase + i * 128, 8)
            pltpu.sync_copy(dA_hbm.at[t, i], dslot)
            pltpu.sync_copy(c_hbm.at[c, pl.ds(off, 128)], cslot)

        _load(0, didx0, cbuf)

        @pl.loop(0, SNC // 2)
        def _(j):
            a = 2 * j
            pltpu.async_copy(cbuf, acc_sh.at[didx0], sem0, add=True)
            _load(a + 1, didx1, cbuf1)
            pltpu.async_copy(cbuf1, acc_sh.at[didx1], sem1, add=True)
            pltpu.make_async_copy(cbuf, acc_sh.at[didx0], sem0).wait()

            @pl.when(a + 2 < SNC)
            def _():
                _load(a + 2, didx0, cbuf)
            pltpu.make_async_copy(cbuf1, acc_sh.at[didx1], sem1).wait()

        toff = pl.multiple_of(ebase + SNC * 128, 8)
        pltpu.sync_copy(c_hbm.at[c, pl.ds(toff, 16)], cbuf.at[pl.ds(0, 16)])
        pltpu.sync_copy(cbuf.at[pl.ds(0, 16)], acc_sh.at[didxb], add=True)

        plsc.subcore_barrier()

        @pl.loop(0, nper)
        def _(j):
            kk = t + j * NS

            @pl.when(kk < nfull)
            def _():
                off = pl.multiple_of(kk * 128, 8)
                pltpu.sync_copy(acc_sh.at[pl.ds(off, 128)], cbuf)
                pltpu.sync_copy(cbuf, acc_hbm.at[c, pl.ds(off, 128)])

        @pl.when(t == NS - 1)
        def _():
            pltpu.sync_copy(acc_sh.at[pl.ds(nfull * 128, 16)],
                            cbuf.at[pl.ds(0, 16)])
            pltpu.sync_copy(cbuf.at[pl.ds(0, 16)],
                            acc_hbm.at[c, pl.ds(nfull * 128, 16)])

    return k(x, dstA, dstB)


def _blockdiag(a):
    z = jnp.zeros((H, DH, H, DH), a.dtype)
    idx = jnp.arange(H)
    z = z.at[idx, :, idx, :].set(a)
    return z.reshape(D, D)


_G = np.zeros((D, H), np.float32)
for _i in range(D):
    _G[_i, _i // DH] = 1.0

EDGE_TYPES = (('author', 'writes', 'paper'), ('paper', 'rev_writes', 'author'))


def kernel(h_author, h_paper, edge_index_writes, edge_index_rev_writes,
           edge_time_writes, edge_time_rev_writes, params):
    p = params
    g = jnp.asarray(_G)
    gt = jnp.asarray(_G.T)

    et = {'writes': edge_time_writes, 'rev_writes': edge_time_rev_writes}
    ei = {'writes': edge_index_writes, 'rev_writes': edge_index_rev_writes}
    gidx = {}
    sidx = {}
    for (_s, r, _d) in EDGE_TYPES:
        src2 = ei[r][0].reshape(NC * NS, EPW)
        dst2 = ei[r][1].reshape(NC * NS, EPW)
        gidx[r] = (src2[:, :GNC * 128].reshape(NC * NS, GNC, 128),
                   src2[:, EPW - 16:],
                   dst2[:, :GNC * 128].reshape(NC * NS, GNC, 128),
                   dst2[:, EPW - 16:])
        dstt = ei[r][1].reshape(NS, EPT)
        sidx[r] = (dstt[:, :SNC * 128].reshape(NS, SNC, 128),
                   dstt[:, SNC * 128:])

    ea = {}
    for (_s, r, _d) in EDGE_TYPES:
        ea[r] = _edge_attr(et[r], p['edge_lin_' + r + '_w'],
                           p['edge_lin_' + r + '_b'])

    h = {'author': h_author, 'paper': h_paper}
    src_rel = {'author': 'writes', 'paper': 'rev_writes'}

    for l in range(2):
        qkv = {}
        for t in ('author', 'paper'):
            r = src_rel[t]
            a_bd = _blockdiag(p['L%d_arel_%s' % (l, r)])
            m_bd = _blockdiag(p['L%d_mrel_%s' % (l, r)])
            qkv[t] = _proj(
                h[t],
                p['L%d_q_%s_w' % (l, t)], p['L%d_q_%s_b' % (l, t)],
                p['L%d_k_%s_w' % (l, t)], p['L%d_k_%s_b' % (l, t)],
                p['L%d_v_%s_w' % (l, t)], p['L%d_v_%s_b' % (l, t)],
                a_bd, m_bd)

        agg = {}
        for (s, r, d) in EDGE_TYPES:
            kve, qe = _gather(qkv[s][1], qkv[d][0], *gidx[r])
            scale = (jnp.repeat(p['L%d_prel_%s' % (l, r)], DH)
                     / math.sqrt(DH)).reshape(1, D)
            contrib, wrep = _edge_math(qe, kve, ea[r], scale, g, gt)
            agg[d] = (_scatter(contrib, *sidx[r]),
                      _scatter(wrep, *sidx[r]))

        newh = {}
        for t in ('author', 'paper'):
            acc, sacc = agg[t]
            newh[t] = _final(
                acc, sacc, h[t],
                p['L%d_a_%s_w' % (l, t)], p['L%d_a_%s_b' % (l, t)],
                p['L%d_skip_%s' % (l, t)],
                p['L%d_ln_%s_g' % (l, t)], p['L%d_ln_%s_b' % (l, t)])
        h = newh

    return (h['author'], h['paper'])


# speedup vs baseline: 1.0577x; 1.0577x over previous
import functools
import math

import jax
import jax.numpy as jnp
import numpy as np
from jax import lax
from jax.experimental import pallas as pl
from jax.experimental.pallas import tpu as pltpu
from jax.experimental.pallas import tpu_sc as plsc

D = 256
H = 8
DH = D // H
N = 10000
E = 160000

NC = 2
NS = 16
EPW = E // (NC * NS)
GC = 40
SCC = 200
EPT = E // NS
NCH = EPT // SCC
RPT = N // NS

@functools.cache
def _mesh():
    return plsc.VectorSubcoreMesh(core_axis_name="c", subcore_axis_name="s")


def _ea_body(et_ref, w_ref, b_ref, o_ref):
    o_ref[...] = jnp.dot(et_ref[...], w_ref[...],
                         preferred_element_type=jnp.float32) + b_ref[...]


def _edge_attr(et, w, b):
    BE = 2000
    return pl.pallas_call(
        _ea_body,
        grid=(E // BE,),
        in_specs=[pl.BlockSpec((BE, 16), lambda i: (i, 0)),
                  pl.BlockSpec((16, D), lambda i: (0, 0)),
                  pl.BlockSpec((1, D), lambda i: (0, 0))],
        out_specs=pl.BlockSpec((BE, D), lambda i: (i, 0)),
        out_shape=jax.ShapeDtypeStruct((E, D), jnp.float32),
    )(et, w, b.reshape(1, D))


def _proj_body(h_ref, wq_ref, bq_ref, wk_ref, bk_ref, wv_ref, bv_ref,
               a_ref, m_ref, q_ref, kv_ref):
    h = h_ref[...]
    q_ref[...] = jnp.dot(h, wq_ref[...],
                         preferred_element_type=jnp.float32) + bq_ref[...]
    k = jnp.dot(h, wk_ref[...], preferred_element_type=jnp.float32) + bk_ref[...]
    v = jnp.dot(h, wv_ref[...], preferred_element_type=jnp.float32) + bv_ref[...]
    kv_ref[:, :D] = jnp.dot(k, a_ref[...], preferred_element_type=jnp.float32)
    kv_ref[:, D:] = jnp.dot(v, m_ref[...], preferred_element_type=jnp.float32)


def _proj(h, wq, bq, wk, bk, wv, bv, a_bd, m_bd):
    BN = 1000
    return pl.pallas_call(
        _proj_body,
        grid=(N // BN,),
        in_specs=[pl.BlockSpec((BN, D), lambda i: (i, 0))] +
                 [pl.BlockSpec((D, D), lambda i: (0, 0)),
                  pl.BlockSpec((1, D), lambda i: (0, 0))] * 3 +
                 [pl.BlockSpec((D, D), lambda i: (0, 0)),
                  pl.BlockSpec((D, D), lambda i: (0, 0))],
        out_specs=[pl.BlockSpec((BN, D), lambda i: (i, 0)),
                   pl.BlockSpec((BN, 2 * D), lambda i: (i, 0))],
        out_shape=[jax.ShapeDtypeStruct((N, D), jnp.float32),
                   jax.ShapeDtypeStruct((N, 2 * D), jnp.float32)],
    )(h, wq, bq.reshape(1, D), wk, bk.reshape(1, D), wv, bv.reshape(1, D),
      a_bd, m_bd)


def _edge_math_body(q_ref, kv_ref, ea_ref, sc_ref, g_ref, gt_ref,
                    contrib_ref, w_ref):
    ea = ea_ref[...]
    kj = kv_ref[:, :D] + ea
    vj = kv_ref[:, D:] + ea
    prod = q_ref[...] * kj * sc_ref[...]
    alpha = jnp.dot(prod, g_ref[...], preferred_element_type=jnp.float32)
    w = jnp.exp(alpha)
    wrep = jnp.dot(w, gt_ref[...], preferred_element_type=jnp.float32)
    c = vj * wrep
    contrib_ref[0] = c[:, :128]
    contrib_ref[1] = c[:, 128:]
    w_ref[0] = wrep[:, :128]
    w_ref[1] = wrep[:, 128:]


def _edge_math(qe, kve, ea, scale, g, gt):
    BE = 2000
    return pl.pallas_call(
        _edge_math_body,
        grid=(E // BE,),
        in_specs=[pl.BlockSpec((BE, D), lambda i: (i, 0)),
                  pl.BlockSpec((BE, 2 * D), lambda i: (i, 0)),
                  pl.BlockSpec((BE, D), lambda i: (i, 0)),
                  pl.BlockSpec((1, D), lambda i: (0, 0)),
                  pl.BlockSpec((D, H), lambda i: (0, 0)),
                  pl.BlockSpec((H, D), lambda i: (0, 0))],
        out_specs=[pl.BlockSpec((NC, BE, 128), lambda i: (0, i, 0)),
                   pl.BlockSpec((NC, BE, 128), lambda i: (0, i, 0))],
        out_shape=[jax.ShapeDtypeStruct((NC, E, 128), jnp.float32),
                   jax.ShapeDtypeStruct((NC, E, 128), jnp.float32)],
    )(qe, kve, ea, scale, g, gt)


def _final_body(acc_ref, s_ref, h_ref, wa_ref, ba_ref, sk_ref,
                g_ref, b_ref, o_ref):
    accc = jnp.concatenate([acc_ref[0], acc_ref[1]], axis=-1)
    srep = jnp.concatenate([s_ref[0], s_ref[1]], axis=-1)
    o1 = accc / (srep + 1e-9)
    g1 = jax.nn.gelu(o1)
    o2 = jnp.dot(g1, wa_ref[...], preferred_element_type=jnp.float32) + ba_ref[...]
    beta = jax.nn.sigmoid(sk_ref[0, 0])
    o3 = beta * o2 + (1.0 - beta) * h_ref[...]
    mu = jnp.mean(o3, axis=-1, keepdims=True)
    xc = o3 - mu
    var = jnp.mean(xc * xc, axis=-1, keepdims=True)
    o_ref[...] = g_ref[...] * (xc / jnp.sqrt(var + 1e-5)) + b_ref[...]


def _final(acc, sacc, h, wa, ba, skip, g, b):
    BN = 1000
    return pl.pallas_call(
        _final_body,
        grid=(N // BN,),
        in_specs=[pl.BlockSpec((NC, BN, 128), lambda i: (0, i, 0)),
                  pl.BlockSpec((NC, BN, 128), lambda i: (0, i, 0)),
                  pl.BlockSpec((BN, D), lambda i: (i, 0)),
                  pl.BlockSpec((D, D), lambda i: (0, 0)),
                  pl.BlockSpec((1, D), lambda i: (0, 0)),
                  pl.BlockSpec((1, 1), lambda i: (0, 0)),
                  pl.BlockSpec((1, D), lambda i: (0, 0)),
                  pl.BlockSpec((1, D), lambda i: (0, 0))],
        out_specs=pl.BlockSpec((BN, D), lambda i: (i, 0)),
        out_shape=jax.ShapeDtypeStruct((N, D), jnp.float32),
    )(acc, sacc, h, wa, ba.reshape(1, D), skip.reshape(1, 1),
      g.reshape(1, D), b.reshape(1, D))


GNC = EPW // 128
GTL = EPW - GNC * 128


def _gather(kv, q, srcA, srcB, dstA, dstB):
    @functools.partial(
        pl.kernel,
        out_type=[jax.ShapeDtypeStruct((E, 2 * D), jnp.float32),
                  jax.ShapeDtypeStruct((E, D), jnp.float32)],
        mesh=_mesh(),
        scratch_types=[pltpu.VMEM((GNC, 128), jnp.int32),
                       pltpu.VMEM((GNC, 128), jnp.int32),
                       pltpu.VMEM((16,), jnp.int32),
                       pltpu.VMEM((16,), jnp.int32),
                       pltpu.VMEM((128, 2 * D), jnp.float32),
                       pltpu.VMEM((128, D), jnp.float32),
                       pltpu.SemaphoreType.DMA,
                       pltpu.SemaphoreType.DMA,
                       pltpu.SemaphoreType.DMA,
                       pltpu.SemaphoreType.DMA],
    )
    def k(kv_hbm, q_hbm, srcA_hbm, srcB_hbm, dstA_hbm, dstB_hbm,
          kve_hbm, qe_hbm, sidx, didx, sidxb, didxb, kvbuf, qbuf,
          wsem_kv, wsem_q, gsem_kv, gsem_q):
        wid = lax.axis_index("c") * NS + lax.axis_index("s")
        base = pl.multiple_of(wid * EPW, 8)
        pltpu.sync_copy(srcA_hbm.at[wid], sidx)
        pltpu.sync_copy(dstA_hbm.at[wid], didx)
        pltpu.sync_copy(srcB_hbm.at[wid], sidxb)
        pltpu.sync_copy(dstB_hbm.at[wid], didxb)

        @pl.loop(0, GNC)
        def _(i):
            off = pl.multiple_of(i * 128, 8)

            @pl.when(i > 0)
            def _():
                pltpu.make_async_copy(
                    kvbuf, kve_hbm.at[pl.ds(base, 128)], wsem_kv).wait()
                pltpu.make_async_copy(
                    qbuf, qe_hbm.at[pl.ds(base, 128)], wsem_q).wait()
            pltpu.async_copy(kv_hbm.at[sidx.at[i]], kvbuf, gsem_kv)
            pltpu.async_copy(q_hbm.at[didx.at[i]], qbuf, gsem_q)
            pltpu.make_async_copy(kv_hbm.at[sidx.at[i]], kvbuf, gsem_kv).wait()
            pltpu.async_copy(kvbuf, kve_hbm.at[pl.ds(base + off, 128)],
                             wsem_kv)
            pltpu.make_async_copy(q_hbm.at[didx.at[i]], qbuf, gsem_q).wait()
            pltpu.async_copy(qbuf, qe_hbm.at[pl.ds(base + off, 128)], wsem_q)

        pltpu.make_async_copy(kvbuf, kve_hbm.at[pl.ds(base, 128)],
                              wsem_kv).wait()
        pltpu.make_async_copy(qbuf, qe_hbm.at[pl.ds(base, 128)],
                              wsem_q).wait()

        tb = pl.multiple_of(base + EPW - 16, 8)
        pltpu.sync_copy(kv_hbm.at[sidxb], kvbuf.at[pl.ds(0, 16)])
        pltpu.sync_copy(kvbuf.at[pl.ds(0, 16)], kve_hbm.at[pl.ds(tb, 16)])
        pltpu.sync_copy(q_hbm.at[didxb], qbuf.at[pl.ds(0, 16)])
        pltpu.sync_copy(qbuf.at[pl.ds(0, 16)], qe_hbm.at[pl.ds(tb, 16)])

    return k(kv, q, srcA, srcB, dstA, dstB)


SNC = EPT // 128
STL = EPT - SNC * 128


def _scatter(x, dstA, dstB):
    @functools.partial(
        pl.kernel,
        out_type=jax.ShapeDtypeStruct((NC, N, 128), jnp.float32),
        mesh=_mesh(),
        scratch_types=[pltpu.VMEM((128,), jnp.int32),
                       pltpu.VMEM((128,), jnp.int32),
                       pltpu.VMEM((16,), jnp.int32),
                       pltpu.VMEM((128, 128), jnp.float32),
                       pltpu.VMEM((128, 128), jnp.float32),
                       pltpu.VMEM_SHARED((N, 128), jnp.float32),
                       pltpu.SemaphoreType.DMA,
                       pltpu.SemaphoreType.DMA,
                       pltpu.SemaphoreType.DMA],
    )
    def k(c_hbm, dA_hbm, dB_hbm, acc_hbm, didx0, didx1, didxb,
          cbuf, cbuf1, acc_sh, sem0, sem1, lsem):
        c = lax.axis_index("c")
        t = lax.axis_index("s")
        nfull = N // 128
        nper = (nfull + NS - 1) // NS
        pltpu.sync_copy(dB_hbm.at[t], didxb)

        @pl.loop(0, 128)
        def _(r):
            @pl.loop(0, 8)
            def _(j):
                cbuf[r, pl.ds(j * 16, 16)] = jnp.zeros((16,), jnp.float32)

        @pl.loop(0, nper)
        def _(j):
            kk = t + j * NS

            @pl.when(kk < nfull)
            def _():
                off = pl.multiple_of(kk * 128, 8)
                pltpu.sync_copy(cbuf, acc_sh.at[pl.ds(off, 128)])

        @pl.when(t == NS - 1)
        def _():
            pltpu.sync_copy(cbuf.at[pl.ds(0, 16)],
                            acc_sh.at[pl.ds(nfull * 128, 16)])
        plsc.subcore_barrier()

        ebase = t * EPT

        def _load(i, dslot, cslot):
            off = pl.multiple_of(ebase + i * 128, 8)
            pltpu.async_copy(dA_hbm.at[t, i], dslot, lsem)
            pltpu.async_copy(c_hbm.at[c, pl.ds(off, 128)], cslot, lsem)
            pltpu.make_async_copy(dA_hbm.at[t, i], dslot, lsem).wait()
            pltpu.make_async_copy(c_hbm.at[c, pl.ds(off, 128)], cslot,
                                  lsem).wait()

        _load(0, didx0, cbuf)

        @pl.loop(0, SNC // 2)
        def _(j):
            a = 2 * j
            pltpu.async_copy(cbuf, acc_sh.at[didx0], sem0, add=True)
            _load(a + 1, didx1, cbuf1)
            pltpu.async_copy(cbuf1, acc_sh.at[didx1], sem1, add=True)
            pltpu.make_async_copy(cbuf, acc_sh.at[didx0], sem0).wait()

            @pl.when(a + 2 < SNC)
            def _():
                _load(a + 2, didx0, cbuf)
            pltpu.make_async_copy(cbuf1, acc_sh.at[didx1], sem1).wait()

        toff = pl.multiple_of(ebase + SNC * 128, 8)
        pltpu.sync_copy(c_hbm.at[c, pl.ds(toff, 16)], cbuf.at[pl.ds(0, 16)])
        pltpu.sync_copy(cbuf.at[pl.ds(0, 16)], acc_sh.at[didxb], add=True)

        plsc.subcore_barrier()

        @pl.loop(0, nper)
        def _(j):
            kk = t + j * NS

            @pl.when(kk < nfull)
            def _():
                off = pl.multiple_of(kk * 128, 8)
                pltpu.sync_copy(acc_sh.at[pl.ds(off, 128)], cbuf)
                pltpu.sync_copy(cbuf, acc_hbm.at[c, pl.ds(off, 128)])

        @pl.when(t == NS - 1)
        def _():
            pltpu.sync_copy(acc_sh.at[pl.ds(nfull * 128, 16)],
                            cbuf.at[pl.ds(0, 16)])
            pltpu.sync_copy(cbuf.at[pl.ds(0, 16)],
                            acc_hbm.at[c, pl.ds(nfull * 128, 16)])

    return k(x, dstA, dstB)


def _blockdiag(a):
    z = jnp.zeros((H, DH, H, DH), a.dtype)
    idx = jnp.arange(H)
    z = z.at[idx, :, idx, :].set(a)
    return z.reshape(D, D)


_G = np.zeros((D, H), np.float32)
for _i in range(D):
    _G[_i, _i // DH] = 1.0

EDGE_TYPES = (('author', 'writes', 'paper'), ('paper', 'rev_writes', 'author'))


def kernel(h_author, h_paper, edge_index_writes, edge_index_rev_writes,
           edge_time_writes, edge_time_rev_writes, params):
    p = params
    g = jnp.asarray(_G)
    gt = jnp.asarray(_G.T)

    et = {'writes': edge_time_writes, 'rev_writes': edge_time_rev_writes}
    ei = {'writes': edge_index_writes, 'rev_writes': edge_index_rev_writes}
    gidx = {}
    sidx = {}
    for (_s, r, _d) in EDGE_TYPES:
        src2 = ei[r][0].reshape(NC * NS, EPW)
        dst2 = ei[r][1].reshape(NC * NS, EPW)
        gidx[r] = (src2[:, :GNC * 128].reshape(NC * NS, GNC, 128),
                   src2[:, EPW - 16:],
                   dst2[:, :GNC * 128].reshape(NC * NS, GNC, 128),
                   dst2[:, EPW - 16:])
        dstt = ei[r][1].reshape(NS, EPT)
        sidx[r] = (dstt[:, :SNC * 128].reshape(NS, SNC, 128),
                   dstt[:, SNC * 128:])

    ea = {}
    for (_s, r, _d) in EDGE_TYPES:
        ea[r] = _edge_attr(et[r], p['edge_lin_' + r + '_w'],
                           p['edge_lin_' + r + '_b'])

    h = {'author': h_author, 'paper': h_paper}
    src_rel = {'author': 'writes', 'paper': 'rev_writes'}

    for l in range(2):
        qkv = {}
        for t in ('author', 'paper'):
            r = src_rel[t]
            a_bd = _blockdiag(p['L%d_arel_%s' % (l, r)])
            m_bd = _blockdiag(p['L%d_mrel_%s' % (l, r)])
            qkv[t] = _proj(
                h[t],
                p['L%d_q_%s_w' % (l, t)], p['L%d_q_%s_b' % (l, t)],
                p['L%d_k_%s_w' % (l, t)], p['L%d_k_%s_b' % (l, t)],
                p['L%d_v_%s_w' % (l, t)], p['L%d_v_%s_b' % (l, t)],
                a_bd, m_bd)

        agg = {}
        for (s, r, d) in EDGE_TYPES:
            kve, qe = _gather(qkv[s][1], qkv[d][0], *gidx[r])
            scale = (jnp.repeat(p['L%d_prel_%s' % (l, r)], DH)
                     / math.sqrt(DH)).reshape(1, D)
            contrib, wrep = _edge_math(qe, kve, ea[r], scale, g, gt)
            agg[d] = (_scatter(contrib, *sidx[r]),
                      _scatter(wrep, *sidx[r]))

        newh = {}
        for t in ('author', 'paper'):
            acc, sacc = agg[t]
            newh[t] = _final(
                acc, sacc, h[t],
                p['L%d_a_%s_w' % (l, t)], p['L%d_a_%s_b' % (l, t)],
                p['L%d_skip_%s' % (l, t)],
                p['L%d_ln_%s_g' % (l, t)], p['L%d_ln_%s_b' % (l, t)])
        h = newh

    return (h['author'], h['paper'])
